# baseline (speedup 1.0000x reference)
import functools
import jax
import jax.numpy as jnp
from jax import lax
from jax.experimental import pallas as pl
from jax.experimental.pallas import tpu as pltpu
from jax.experimental.pallas import tpu_sc as plsc

N = 10000
NP = 10240
E = 320000
EROWS = 2560
EP = EROWS * 128
DF = 128
GH = 32
B = 100
T = 20
BP = 128
LH = 128
NCLS = 10

NCORES = 2
NSUB = 16
RPT = NP // NSUB
ERW = EROWS // (NCORES * NSUB)
IB = 8

@functools.cache
def _mesh():
    return plsc.VectorSubcoreMesh(
        core_axis_name="c", subcore_axis_name="s", num_cores=NCORES, num_subcores=NSUB
    )


def _hist_body(col_hbm, out_hbm, idx_v, ones_v, buf_v, hist_sp, semi, sems):
    cid = lax.axis_index("c")
    sid = lax.axis_index("s")
    wid = sid * NCORES + cid
    idone = pltpu.async_copy(col_hbm.at[pl.ds(wid * ERW, ERW)], idx_v, semi)
    for i in range(8):
        ones_v[pl.ds(i * 16, 16)] = jnp.ones((16,), jnp.float32)
    for i in range(RPT // 16):
        buf_v[pl.ds(i * 16, 16)] = jnp.zeros((16,), jnp.float32)
    pltpu.sync_copy(buf_v, hist_sp.at[pl.ds(sid * RPT, RPT)])
    idone.wait()
    plsc.subcore_barrier()
    sd = {}
    for k in range(ERW):
        sd[k] = pltpu.async_copy(ones_v, hist_sp.at[idx_v.at[k]], sems, add=True)
        if k >= 8:
            sd[k - 8].wait()
    for k in range(ERW - 8, ERW):
        sd[k].wait()
    plsc.subcore_barrier()
    pltpu.sync_copy(hist_sp.at[pl.ds(sid * RPT, RPT)], buf_v)
    pltpu.sync_copy(buf_v, out_hbm.at[cid, pl.ds(sid * RPT, RPT)])


@functools.cache
def _hist_call():
    return pl.kernel(
        _hist_body,
        out_type=jax.ShapeDtypeStruct((NCORES, NP), jnp.float32),
        mesh=_mesh(),
        compiler_params=pltpu.CompilerParams(use_tc_tiling_on_sc=False),
        scratch_types=[
            pltpu.VMEM((ERW, 128), jnp.int32),
            pltpu.VMEM((128,), jnp.float32),
            pltpu.VMEM((RPT,), jnp.float32),
            pltpu.VMEM_SHARED((NP,), jnp.float32),
            pltpu.SemaphoreType.DMA,
            pltpu.SemaphoreType.DMA,
        ],
    )


NBUF = 8
LAG = 4
ACC3 = 5 * 2560
RPT3 = ACC3 // NSUB


def _make_agg_body(acc_rows, rpt_out, permuted):
    def body(table_hbm, row_hbm, col_hbm, zeros_hbm, perm_hbm, out_hbm,
             idxr_v, idxc_v, rows_v, buf_v, buft_v, pidx_v, acc_sp,
             semi, semg, sems):
        cid = lax.axis_index("c")
        sid = lax.axis_index("s")
        wid = sid * NCORES + cid
        ir = pltpu.async_copy(row_hbm.at[pl.ds(wid * ERW, ERW)], idxr_v, semi)
        ic = pltpu.async_copy(col_hbm.at[pl.ds(wid * ERW, ERW)], idxc_v, semi)
        pltpu.sync_copy(table_hbm.at[pl.ds(sid * RPT, RPT)], buft_v)
        if permuted:
            pltpu.sync_copy(zeros_hbm.at[pl.ds(sid * rpt_out, rpt_out)], buf_v)
            pltpu.sync_copy(buf_v, acc_sp.at[pl.ds(sid * rpt_out, rpt_out)])
        else:
            @pl.when(cid == 0)
            def _():
                pltpu.sync_copy(buft_v, acc_sp.at[pl.ds(sid * RPT, RPT)])

            @pl.when(cid == 1)
            def _():
                pltpu.sync_copy(zeros_hbm.at[pl.ds(sid * RPT, RPT)], buf_v)
                pltpu.sync_copy(buf_v, acc_sp.at[pl.ds(sid * RPT, RPT)])
        ir.wait()
        ic.wait()
        plsc.subcore_barrier()
        if permuted:
            @pl.when(cid == 0)
            def _():
                pltpu.sync_copy(perm_hbm.at[pl.ds(sid * 5, 5)], pidx_v)
                for q in range(5):
                    pltpu.sync_copy(buft_v.at[pl.ds(q * 128, 128)],
                                    acc_sp.at[pidx_v.at[q]], add=True)
        gd = {}
        sd = {}

        def scatter(j):
            gd[j].wait()
            sd[j] = pltpu.async_copy(
                rows_v.at[j % NBUF], acc_sp.at[idxc_v.at[j]], sems, add=True)

        for k in range(ERW):
            if k >= NBUF:
                sd[k - NBUF].wait()
            gd[k] = pltpu.async_copy(
                table_hbm.at[idxr_v.at[k]], rows_v.at[k % NBUF], semg)
            if k >= LAG:
                scatter(k - LAG)
        for j in range(ERW - LAG, ERW):
            scatter(j)
        for j in range(ERW - NBUF, ERW):
            sd[j].wait()
        plsc.subcore_barrier()
        pltpu.sync_copy(acc_sp.at[pl.ds(sid * rpt_out, rpt_out)], buf_v)
        pltpu.sync_copy(buf_v, out_hbm.at[cid, pl.ds(sid * rpt_out, rpt_out)])
    return body


@functools.cache
def _agg_call(permuted):
    acc_rows = ACC3 if permuted else NP
    rpt_out = RPT3 if permuted else RPT
    return pl.kernel(
        _make_agg_body(acc_rows, rpt_out, permuted),
        out_type=jax.ShapeDtypeStruct((NCORES, acc_rows, GH), jnp.float32),
        mesh=_mesh(),
        compiler_params=pltpu.CompilerParams(use_tc_tiling_on_sc=False),
        scratch_types=[
            pltpu.VMEM((ERW, 128), jnp.int32),
            pltpu.VMEM((ERW, 128), jnp.int32),
            pltpu.VMEM((NBUF, 128, GH), jnp.float32),
            pltpu.VMEM((rpt_out, GH), jnp.float32),
            pltpu.VMEM((RPT, GH), jnp.float32),
            pltpu.VMEM((5, 128), jnp.int32),
            pltpu.VMEM_SHARED((acc_rows, GH), jnp.float32),
            pltpu.SemaphoreType.DMA,
            pltpu.SemaphoreType.DMA,
            pltpu.SemaphoreType.DMA,
        ],
    )


def _prep_body(hist_ref, x_ref, w1_ref, dis_ref, yhat_ref):
    h = hist_ref[0] + hist_ref[1]
    dis = lax.rsqrt(h + 1.0)
    dis_ref[...] = dis
    y = jnp.dot(x_ref[...], w1_ref[...], preferred_element_type=jnp.float32)
    yhat_ref[0:N] = y * dis[0:N]
    yhat_ref[N:NP] = jnp.zeros((NP - N, GH), jnp.float32)


def _prep_call(hist3, x, w1):
    return pl.pallas_call(
        _prep_body,
        out_shape=[
            jax.ShapeDtypeStruct((NP, 1), jnp.float32),
            jax.ShapeDtypeStruct((NP, GH), jnp.float32),
        ],
    )(hist3, x, w1)


def _mid_body(p_ref, dis_ref, b_ref, w_ref, out_ref):
    dis = dis_ref[...]
    s = (p_ref[0] + p_ref[1]) * dis + b_ref[...]
    h = jnp.maximum(s, 0.0)
    out_ref[...] = jnp.dot(h, w_ref[...], preferred_element_type=jnp.float32) * dis


def _mid_call(p, dis, b, w):
    return pl.pallas_call(
        _mid_body,
        out_shape=jax.ShapeDtypeStruct((NP, GH), jnp.float32),
    )(p, dis, b, w)


def _gates(g, c):
    ig = jax.nn.sigmoid(g[:, 0:LH])
    fg = jax.nn.sigmoid(g[:, LH:2 * LH])
    gg = jnp.tanh(g[:, 2 * LH:3 * LH])
    og = jax.nn.sigmoid(g[:, 3 * LH:4 * LH])
    c2 = fg * c + ig * gg
    h2 = og * jnp.tanh(c2)
    return h2, c2


def _lstm_body(p_ref, dis_ref, b3_ref,
               w0f_ref, u0f_ref, b0fi_ref, b0fh_ref,
               w0b_ref, u0b_ref, b0bi_ref, b0bh_ref,
               w1f_ref, u1f_ref, b1fi_ref, b1fh_ref,
               w1b_ref, u1b_ref, b1bi_ref, b1bh_ref,
               fcw_ref, fcb_ref, out_ref,
               pf_ref, pb_ref, out0_ref, pf1_ref):
    b3 = b3_ref[...]
    xs = jnp.concatenate(
        [(p_ref[0, j] + p_ref[1, j]) * dis_ref[j] + b3 for j in range(5)],
        axis=1)
    b0f = b0fi_ref[...] + b0fh_ref[...]
    b0b = b0bi_ref[...] + b0bh_ref[...]
    pf_ref[...] = jnp.dot(xs, w0f_ref[...], preferred_element_type=jnp.float32) + b0f
    pb_ref[...] = jnp.dot(xs, w0b_ref[...], preferred_element_type=jnp.float32) + b0b
    u0f = u0f_ref[...]
    u0b = u0b_ref[...]
    z = jnp.zeros((BP, LH), jnp.float32)

    def body0(t, carry):
        hf, cf, hb, cb = carry
        rf = pl.multiple_of(t * BP, BP)
        gf = pf_ref[pl.ds(rf, BP), :] + jnp.dot(hf, u0f, preferred_element_type=jnp.float32)
        hf, cf = _gates(gf, cf)
        out0_ref[pl.ds(rf, BP), 0:LH] = hf
        rb = pl.multiple_of((T - 1 - t) * BP, BP)
        gb = pb_ref[pl.ds(rb, BP), :] + jnp.dot(hb, u0b, preferred_element_type=jnp.float32)
        hb, cb = _gates(gb, cb)
        out0_ref[pl.ds(rb, BP), LH:2 * LH] = hb
        return hf, cf, hb, cb

    lax.fori_loop(0, T, body0, (z, z, z, z))

    x1 = out0_ref[...]
    b1f = b1fi_ref[...] + b1fh_ref[...]
    pf1_ref[...] = jnp.dot(x1, w1f_ref[...], preferred_element_type=jnp.float32) + b1f
    u1f = u1f_ref[...]

    def body1(t, carry):
        hf, cf = carry
        rf = pl.multiple_of(t * BP, BP)
        gf = pf1_ref[pl.ds(rf, BP), :] + jnp.dot(hf, u1f, preferred_element_type=jnp.float32)
        return _gates(gf, cf)

    hf1, _ = lax.fori_loop(0, T, body1, (z, z))

    b1b = b1bi_ref[...] + b1bh_ref[...]
    x19 = out0_ref[(T - 1) * BP:T * BP, :]
    gb1 = jnp.dot(x19, w1b_ref[...], preferred_element_type=jnp.float32) + b1b
    hb1, _ = _gates(gb1, z)

    feat = jnp.concatenate([hf1, hb1], axis=1)
    out_ref[...] = jnp.dot(feat, fcw_ref[...], preferred_element_type=jnp.float32) + fcb_ref[...]


def _lstm_call(p3, dis5, b3, *args):
    return pl.pallas_call(
        _lstm_body,
        out_shape=jax.ShapeDtypeStruct((BP, NCLS), jnp.float32),
        scratch_shapes=[
            pltpu.VMEM((T * BP, 4 * LH), jnp.float32),
            pltpu.VMEM((T * BP, 4 * LH), jnp.float32),
            pltpu.VMEM((T * BP, 2 * LH), jnp.float32),
            pltpu.VMEM((T * BP, 4 * LH), jnp.float32),
        ],
    )(p3, dis5, b3, *args)


def kernel(x, edge_index, W1, b1, W2, b2, W3, b3,
           Wih0f, Whh0f, bih0f, bhh0f,
           Wih0b, Whh0b, bih0b, bhh0b,
           Wih1f, Whh1f, bih1f, bhh1f,
           Wih1b, Whh1b, bih1b, bhh1b,
           fcW, fcb):
    row = edge_index[0].astype(jnp.int32)
    col = edge_index[1].astype(jnp.int32)
    pad = jnp.full((EP - E,), NP - 1, jnp.int32)
    rowp = jnp.concatenate([row, pad]).reshape(EROWS, 128)
    colp = jnp.concatenate([col, pad]).reshape(EROWS, 128)
    ztbl = jnp.zeros((ACC3, GH), jnp.float32)

    n = jnp.arange(NP, dtype=jnp.int32)
    perm = (n % 5) * (T * BP) + ((n // 5) % T) * BP + n // B
    colp3 = perm[colp]
    perm2d = perm.reshape(NP // 128, 128)
    inv = jnp.zeros((ACC3,), jnp.int32).at[perm].set(n)

    hist = _hist_call()(colp)
    dis, yhat1 = _prep_call(hist.reshape(NCORES, NP, 1), x, W1)
    p1 = _agg_call(False)(yhat1, rowp, colp, ztbl, perm2d)
    yhat2 = _mid_call(p1, dis, b1, W2)
    p2 = _agg_call(False)(yhat2, rowp, colp, ztbl, perm2d)
    yhat3 = _mid_call(p2, dis, b2, W3)
    p3 = _agg_call(True)(yhat3, rowp, colp3, ztbl, perm2d)

    dis5 = dis[inv].reshape(5, T * BP, 1)
    out = _lstm_call(
        p3.reshape(NCORES, 5, T * BP, GH), dis5, b3,
        Wih0f.T, Whh0f.T, bih0f, bhh0f,
        Wih0b.T, Whh0b.T, bih0b, bhh0b,
        Wih1f.T, Whh1f.T, bih1f, bhh1f,
        Wih1b.T, Whh1b.T, bih1b, bhh1b,
        fcW.T, fcb,
    )
    return out[:B]


# speedup vs baseline: 11.9856x; 11.9856x over previous
import functools
import jax
import jax.numpy as jnp
from jax import lax
from jax.experimental import pallas as pl
from jax.experimental.pallas import tpu as pltpu
from jax.experimental.pallas import tpu_sc as plsc

N = 10000
E = 320000
EROWS = 2560
EP = EROWS * 128
DF = 128
GH = 32
B = 100
T = 20
BP = 112
LH = 128
NCLS = 10
NS = 5 * T * BP

NCORES = 2
NSUB = 16
RPT = NS // NSUB
ERW = EROWS // (NCORES * NSUB)
NBUF = 8
LAG = 4
PADSLOT = 4 * T * BP + 19 * BP + 111


@functools.cache
def _mesh():
    return plsc.VectorSubcoreMesh(
        core_axis_name="c", subcore_axis_name="s", num_cores=NCORES, num_subcores=NSUB
    )


def _hist_body(col_hbm, out_hbm, idx_v, hist_v, sem):
    cid = lax.axis_index("c")
    sid = lax.axis_index("s")
    wid = sid * NCORES + cid
    idone = pltpu.async_copy(
        col_hbm.at[pl.ds(wid * ERW * 128, ERW * 128)], idx_v, sem)
    ones16 = jnp.ones((16,), jnp.float32)

    def zbody(i, c):
        hist_v[pl.ds(i * 16, 16)] = jnp.zeros((16,), jnp.float32)
        return c

    lax.fori_loop(0, NS // 16, zbody, 0)
    idone.wait()

    def abody(i, c):
        iv = idx_v[pl.ds(i * 16, 16)]
        plsc.addupdate_scatter(hist_v, [iv], ones16)
        return c

    lax.fori_loop(0, ERW * 128 // 16, abody, 0)
    pltpu.sync_copy(hist_v, out_hbm.at[wid])


@functools.cache
def _hist_call():
    return pl.kernel(
        _hist_body,
        out_type=jax.ShapeDtypeStruct((NCORES * NSUB, NS), jnp.float32),
        mesh=_mesh(),
        compiler_params=pltpu.CompilerParams(
            use_tc_tiling_on_sc=False, needs_layout_passes=False),
        scratch_types=[
            pltpu.VMEM((ERW * 128,), jnp.int32),
            pltpu.VMEM((NS,), jnp.float32),
            pltpu.SemaphoreType.DMA,
        ],
    )


def _agg_body(table_hbm, row_hbm, col_hbm, zeros_hbm, out_hbm,
              idxr_v, idxc_v, rows_v, buf_v, table_sp, acc_sp,
              semi, semg, sems):
    cid = lax.axis_index("c")
    sid = lax.axis_index("s")
    wid = sid * NCORES + cid
    ir = pltpu.async_copy(row_hbm.at[pl.ds(wid * ERW, ERW)], idxr_v, semi)
    ic = pltpu.async_copy(col_hbm.at[pl.ds(wid * ERW, ERW)], idxc_v, semi)
    pltpu.sync_copy(table_hbm.at[pl.ds(sid * RPT, RPT)], buf_v)
    pltpu.sync_copy(buf_v, table_sp.at[pl.ds(sid * RPT, RPT)])

    @pl.when(cid == 0)
    def _():
        pltpu.sync_copy(buf_v, acc_sp.at[pl.ds(sid * RPT, RPT)])

    @pl.when(cid == 1)
    def _():
        pltpu.sync_copy(zeros_hbm.at[pl.ds(sid * RPT, RPT)], buf_v)
        pltpu.sync_copy(buf_v, acc_sp.at[pl.ds(sid * RPT, RPT)])

    ir.wait()
    ic.wait()
    plsc.subcore_barrier()
    gd = {}
    sd = {}

    def scatter(j):
        gd[j].wait()
        sd[j] = pltpu.async_copy(
            rows_v.at[j % NBUF], acc_sp.at[idxc_v.at[j]], sems, add=True)

    for k in range(ERW):
        if k >= NBUF:
            sd[k - NBUF].wait()
        gd[k] = pltpu.async_copy(
            table_sp.at[idxr_v.at[k]], rows_v.at[k % NBUF], semg)
        if k >= LAG:
            scatter(k - LAG)
    for j in range(ERW - LAG, ERW):
        scatter(j)
    for j in range(ERW - NBUF, ERW):
        sd[j].wait()
    plsc.subcore_barrier()
    pltpu.sync_copy(acc_sp.at[pl.ds(sid * RPT, RPT)], buf_v)
    pltpu.sync_copy(buf_v, out_hbm.at[cid, pl.ds(sid * RPT, RPT)])


@functools.cache
def _agg_call():
    return pl.kernel(
        _agg_body,
        out_type=jax.ShapeDtypeStruct((NCORES, NS, GH), jnp.float32),
        mesh=_mesh(),
        compiler_params=pltpu.CompilerParams(use_tc_tiling_on_sc=False),
        scratch_types=[
            pltpu.VMEM((ERW, 128), jnp.int32),
            pltpu.VMEM((ERW, 128), jnp.int32),
            pltpu.VMEM((NBUF, 128, GH), jnp.float32),
            pltpu.VMEM((RPT, GH), jnp.float32),
            pltpu.VMEM_SHARED((NS, GH), jnp.float32),
            pltpu.VMEM_SHARED((NS, GH), jnp.float32),
            pltpu.SemaphoreType.DMA,
            pltpu.SemaphoreType.DMA,
            pltpu.SemaphoreType.DMA,
        ],
    )


def _prep_body(hist_ref, x_ref, w1_ref, dis_ref, yhat_ref):
    dis = lax.rsqrt(hist_ref[...] + 1.0)
    dis_ref[...] = dis
    y = jnp.dot(x_ref[...], w1_ref[...], preferred_element_type=jnp.float32)
    yhat_ref[...] = y * dis


def _prep_call(hsum, x_tm, w1):
    return pl.pallas_call(
        _prep_body,
        out_shape=[
            jax.ShapeDtypeStruct((NS, 1), jnp.float32),
            jax.ShapeDtypeStruct((NS, GH), jnp.float32),
        ],
    )(hsum, x_tm, w1)


def _mid_body(p_ref, dis_ref, b_ref, w_ref, out_ref):
    dis = dis_ref[...]
    s = (p_ref[0] + p_ref[1]) * dis + b_ref[...]
    h = jnp.maximum(s, 0.0)
    out_ref[...] = jnp.dot(h, w_ref[...], preferred_element_type=jnp.float32) * dis


def _mid_call(p, dis, b, w):
    return pl.pallas_call(
        _mid_body,
        out_shape=jax.ShapeDtypeStruct((NS, GH), jnp.float32),
    )(p, dis, b, w)


def _gates(g, c):
    ig = jax.nn.sigmoid(g[:, 0:LH])
    fg = jax.nn.sigmoid(g[:, LH:2 * LH])
    gg = jnp.tanh(g[:, 2 * LH:3 * LH])
    og = jax.nn.sigmoid(g[:, 3 * LH:4 * LH])
    c2 = fg * c + ig * gg
    h2 = og * jnp.tanh(c2)
    return h2, c2


def _lstm_body(p_ref, dis_ref, b3_ref,
               w0f_ref, u0f_ref, b0fi_ref, b0fh_ref,
               w0b_ref, u0b_ref, b0bi_ref, b0bh_ref,
               w1f_ref, u1f_ref, b1fi_ref, b1fh_ref,
               w1b_ref, u1b_ref, b1bi_ref, b1bh_ref,
               fcw_ref, fcb_ref, out_ref,
               pf_ref, pb_ref, out0_ref, pf1_ref):
    b3 = b3_ref[...]
    xs = jnp.concatenate(
        [(p_ref[0, j] + p_ref[1, j]) * dis_ref[j] + b3 for j in range(5)],
        axis=1)
    b0f = b0fi_ref[...] + b0fh_ref[...]
    b0b = b0bi_ref[...] + b0bh_ref[...]
    pf_ref[...] = jnp.dot(xs, w0f_ref[...], preferred_element_type=jnp.float32) + b0f
    pb_ref[...] = jnp.dot(xs, w0b_ref[...], preferred_element_type=jnp.float32) + b0b
    u0f = u0f_ref[...]
    u0b = u0b_ref[...]
    z = jnp.zeros((BP, LH), jnp.float32)

    def body0(t, carry):
        hf, cf, hb, cb = carry
        rf = pl.multiple_of(t * BP, BP)
        gf = pf_ref[pl.ds(rf, BP), :] + jnp.dot(hf, u0f, preferred_element_type=jnp.float32)
        hf, cf = _gates(gf, cf)
        out0_ref[pl.ds(rf, BP), 0:LH] = hf
        rb = pl.multiple_of((T - 1 - t) * BP, BP)
        gb = pb_ref[pl.ds(rb, BP), :] + jnp.dot(hb, u0b, preferred_element_type=jnp.float32)
        hb, cb = _gates(gb, cb)
        out0_ref[pl.ds(rb, BP), LH:2 * LH] = hb
        return hf, cf, hb, cb

    lax.fori_loop(0, T, body0, (z, z, z, z))

    x1 = out0_ref[...]
    b1f = b1fi_ref[...] + b1fh_ref[...]
    pf1_ref[...] = jnp.dot(x1, w1f_ref[...], preferred_element_type=jnp.float32) + b1f
    u1f = u1f_ref[...]

    def body1(t, carry):
        hf, cf = carry
        rf = pl.multiple_of(t * BP, BP)
        gf = pf1_ref[pl.ds(rf, BP), :] + jnp.dot(hf, u1f, preferred_element_type=jnp.float32)
        return _gates(gf, cf)

    hf1, _ = lax.fori_loop(0, T, body1, (z, z))

    b1b = b1bi_ref[...] + b1bh_ref[...]
    x19 = out0_ref[(T - 1) * BP:T * BP, :]
    gb1 = jnp.dot(x19, w1b_ref[...], preferred_element_type=jnp.float32) + b1b
    hb1, _ = _gates(gb1, z)

    feat = jnp.concatenate([hf1, hb1], axis=1)
    out_ref[...] = jnp.dot(feat, fcw_ref[...], preferred_element_type=jnp.float32) + fcb_ref[...]


def _lstm_call(p3, dis5, b3, *args):
    return pl.pallas_call(
        _lstm_body,
        out_shape=jax.ShapeDtypeStruct((BP, NCLS), jnp.float32),
        scratch_shapes=[
            pltpu.VMEM((T * BP, 4 * LH), jnp.float32),
            pltpu.VMEM((T * BP, 4 * LH), jnp.float32),
            pltpu.VMEM((T * BP, 2 * LH), jnp.float32),
            pltpu.VMEM((T * BP, 4 * LH), jnp.float32),
        ],
    )(p3, dis5, b3, *args)


def _to_slot(ix):
    return (ix % 5) * (T * BP) + ((ix // 5) % T) * BP + ix // B


def kernel(x, edge_index, W1, b1, W2, b2, W3, b3,
           Wih0f, Whh0f, bih0f, bhh0f,
           Wih0b, Whh0b, bih0b, bhh0b,
           Wih1f, Whh1f, bih1f, bhh1f,
           Wih1b, Whh1b, bih1b, bhh1b,
           fcW, fcb):
    row = edge_index[0].astype(jnp.int32)
    col = edge_index[1].astype(jnp.int32)
    padv = jnp.full((EP - E,), PADSLOT, jnp.int32)
    rowp = jnp.concatenate([_to_slot(row), padv]).reshape(EROWS, 128)
    colf = jnp.concatenate([_to_slot(col), padv])
    colp = colf.reshape(EROWS, 128)
    x_tm = jnp.pad(x.reshape(B, T, 5, DF).transpose(2, 1, 0, 3),
                   ((0, 0), (0, 0), (0, BP - B), (0, 0))).reshape(NS, DF)
    ztbl = jnp.zeros((NS, GH), jnp.float32)

    hist = _hist_call()(colf)
    dis, yhat1 = _prep_call(hist.sum(axis=0)[:, None], x_tm, W1)

    def step(yhat, wb):
        w, b = wb
        p = _agg_call()(yhat, rowp, colp, ztbl)
        return _mid_call(p, dis, b, w), None

    yhat3, _ = lax.scan(step, yhat1,
                        (jnp.stack([W2, W3]), jnp.stack([b1, b2])))
    p3 = _agg_call()(yhat3, rowp, colp, ztbl)

    out = _lstm_call(
        p3.reshape(NCORES, 5, T * BP, GH), dis.reshape(5, T * BP, 1), b3,
        Wih0f.T, Whh0f.T, bih0f, bhh0f,
        Wih0b.T, Whh0b.T, bih0b, bhh0b,
        Wih1f.T, Whh1f.T, bih1f, bhh1f,
        Wih1b.T, Whh1b.T, bih1b, bhh1b,
        fcW.T, fcb,
    )
    return out[:B]


# speedup vs baseline: 14.3845x; 1.2002x over previous
import functools
import jax
import jax.numpy as jnp
from jax import lax
from jax.experimental import pallas as pl
from jax.experimental.pallas import tpu as pltpu
from jax.experimental.pallas import tpu_sc as plsc

N = 10000
NP = 10240
E = 320000
EROWS = 2560
EP = EROWS * 128
DF = 128
GH = 32
B = 100
T = 20
BP = 128
LH = 128
NCLS = 10

NCORES = 2
NSUB = 16
RPT = NP // NSUB
ERW = EROWS // (NCORES * NSUB)
NBUF = 8
LAG = 4


@functools.cache
def _mesh():
    return plsc.VectorSubcoreMesh(
        core_axis_name="c", subcore_axis_name="s", num_cores=NCORES, num_subcores=NSUB
    )


def _hist_body(col_hbm, out_hbm, idx_v, hist_v, sem):
    cid = lax.axis_index("c")
    sid = lax.axis_index("s")
    wid = sid * NCORES + cid
    idone = pltpu.async_copy(
        col_hbm.at[pl.ds(wid * ERW * 128, ERW * 128)], idx_v, sem)
    ones16 = jnp.ones((16,), jnp.float32)

    def zbody(i, c):
        hist_v[pl.ds(i * 16, 16)] = jnp.zeros((16,), jnp.float32)
        return c

    lax.fori_loop(0, NP // 16, zbody, 0)
    idone.wait()

    def abody(i, c):
        iv = idx_v[pl.ds(i * 16, 16)]
        plsc.addupdate_scatter(hist_v, [iv], ones16)
        return c

    lax.fori_loop(0, ERW * 128 // 16, abody, 0)
    pltpu.sync_copy(hist_v, out_hbm.at[wid])


@functools.cache
def _hist_call():
    return pl.kernel(
        _hist_body,
        out_type=jax.ShapeDtypeStruct((NCORES * NSUB, NP), jnp.float32),
        mesh=_mesh(),
        compiler_params=pltpu.CompilerParams(
            use_tc_tiling_on_sc=False, needs_layout_passes=False),
        scratch_types=[
            pltpu.VMEM((ERW * 128,), jnp.int32),
            pltpu.VMEM((NP,), jnp.float32),
            pltpu.SemaphoreType.DMA,
        ],
    )


def _agg_body(table_hbm, row_hbm, col_hbm, zeros_hbm, out_hbm,
              idxr_v, idxc_v, rows_v, buf_v, table_sp, acc_sp,
              semi, semg, sems):
    cid = lax.axis_index("c")
    sid = lax.axis_index("s")
    wid = sid * NCORES + cid
    ir = pltpu.async_copy(row_hbm.at[pl.ds(wid * ERW, ERW)], idxr_v, semi)
    ic = pltpu.async_copy(col_hbm.at[pl.ds(wid * ERW, ERW)], idxc_v, semi)
    pltpu.sync_copy(table_hbm.at[pl.ds(sid * RPT, RPT)], buf_v)
    pltpu.sync_copy(buf_v, table_sp.at[pl.ds(sid * RPT, RPT)])

    @pl.when(cid == 0)
    def _():
        pltpu.sync_copy(buf_v, acc_sp.at[pl.ds(sid * RPT, RPT)])

    @pl.when(cid == 1)
    def _():
        pltpu.sync_copy(zeros_hbm.at[pl.ds(sid * RPT, RPT)], buf_v)
        pltpu.sync_copy(buf_v, acc_sp.at[pl.ds(sid * RPT, RPT)])

    ir.wait()
    ic.wait()
    plsc.subcore_barrier()
    gd = {}
    sd = {}

    def scatter(j):
        gd[j].wait()
        sd[j] = pltpu.async_copy(
            rows_v.at[j % NBUF], acc_sp.at[idxc_v.at[j]], sems, add=True)

    for k in range(ERW):
        if k >= NBUF:
            sd[k - NBUF].wait()
        gd[k] = pltpu.async_copy(
            table_sp.at[idxr_v.at[k]], rows_v.at[k % NBUF], semg)
        if k >= LAG:
            scatter(k - LAG)
    for j in range(ERW - LAG, ERW):
        scatter(j)
    for j in range(ERW - NBUF, ERW):
        sd[j].wait()
    plsc.subcore_barrier()
    pltpu.sync_copy(acc_sp.at[pl.ds(sid * RPT, RPT)], buf_v)
    pltpu.sync_copy(buf_v, out_hbm.at[cid, pl.ds(sid * RPT, RPT)])


@functools.cache
def _agg_call():
    return pl.kernel(
        _agg_body,
        out_type=jax.ShapeDtypeStruct((NCORES, NP, GH), jnp.float32),
        mesh=_mesh(),
        compiler_params=pltpu.CompilerParams(use_tc_tiling_on_sc=False),
        scratch_types=[
            pltpu.VMEM((ERW, 128), jnp.int32),
            pltpu.VMEM((ERW, 128), jnp.int32),
            pltpu.VMEM((NBUF, 128, GH), jnp.float32),
            pltpu.VMEM((RPT, GH), jnp.float32),
            pltpu.VMEM_SHARED((NP, GH), jnp.float32),
            pltpu.VMEM_SHARED((NP, GH), jnp.float32),
            pltpu.SemaphoreType.DMA,
            pltpu.SemaphoreType.DMA,
            pltpu.SemaphoreType.DMA,
        ],
    )


def _prep_body(hist_ref, x_ref, w1_ref, dis_ref, yhat_ref):
    dis = lax.rsqrt(hist_ref[...] + 1.0)
    dis_ref[...] = dis
    y = jnp.dot(x_ref[...], w1_ref[...], preferred_element_type=jnp.float32)
    yhat_ref[0:N] = y * dis[0:N]
    yhat_ref[N:NP] = jnp.zeros((NP - N, GH), jnp.float32)


def _prep_call(hsum, x, w1):
    return pl.pallas_call(
        _prep_body,
        out_shape=[
            jax.ShapeDtypeStruct((NP, 1), jnp.float32),
            jax.ShapeDtypeStruct((NP, GH), jnp.float32),
        ],
    )(hsum, x, w1)


def _mid_body(p_ref, dis_ref, b_ref, w_ref, out_ref):
    dis = dis_ref[...]
    s = (p_ref[0] + p_ref[1]) * dis + b_ref[...]
    h = jnp.maximum(s, 0.0)
    out_ref[...] = jnp.dot(h, w_ref[...], preferred_element_type=jnp.float32) * dis


def _mid_call(p, dis, b, w):
    return pl.pallas_call(
        _mid_body,
        out_shape=jax.ShapeDtypeStruct((NP, GH), jnp.float32),
    )(p, dis, b, w)


def _last_body(p_ref, dis_ref, b_ref, out_ref):
    out_ref[...] = (p_ref[0] + p_ref[1]) * dis_ref[...] + b_ref[...]


def _last_call(p, dis, b):
    return pl.pallas_call(
        _last_body,
        out_shape=jax.ShapeDtypeStruct((NP, GH), jnp.float32),
    )(p, dis, b)


def _gates(g, c):
    ig = jax.nn.sigmoid(g[:, 0:LH])
    fg = jax.nn.sigmoid(g[:, LH:2 * LH])
    gg = jnp.tanh(g[:, 2 * LH:3 * LH])
    og = jax.nn.sigmoid(g[:, 3 * LH:4 * LH])
    c2 = fg * c + ig * gg
    h2 = og * jnp.tanh(c2)
    return h2, c2


def _lstm_body(xs_ref, w0f_ref, u0f_ref, b0fi_ref, b0fh_ref,
               w0b_ref, u0b_ref, b0bi_ref, b0bh_ref,
               w1f_ref, u1f_ref, b1fi_ref, b1fh_ref,
               w1b_ref, u1b_ref, b1bi_ref, b1bh_ref,
               fcw_ref, fcb_ref, out_ref,
               pf_ref, pb_ref, out0_ref, pf1_ref):
    xs = xs_ref[...]
    b0f = b0fi_ref[...] + b0fh_ref[...]
    b0b = b0bi_ref[...] + b0bh_ref[...]
    pf_ref[...] = jnp.dot(xs, w0f_ref[...], preferred_element_type=jnp.float32) + b0f
    pb_ref[...] = jnp.dot(xs, w0b_ref[...], preferred_element_type=jnp.float32) + b0b
    u0f = u0f_ref[...]
    u0b = u0b_ref[...]
    z = jnp.zeros((BP, LH), jnp.float32)

    def body0(t, carry):
        hf, cf, hb, cb = carry
        rf = pl.multiple_of(t * BP, BP)
        gf = pf_ref[pl.ds(rf, BP), :] + jnp.dot(hf, u0f, preferred_element_type=jnp.float32)
        hf, cf = _gates(gf, cf)
        out0_ref[pl.ds(rf, BP), 0:LH] = hf
        rb = pl.multiple_of((T - 1 - t) * BP, BP)
        gb = pb_ref[pl.ds(rb, BP), :] + jnp.dot(hb, u0b, preferred_element_type=jnp.float32)
        hb, cb = _gates(gb, cb)
        out0_ref[pl.ds(rb, BP), LH:2 * LH] = hb
        return hf, cf, hb, cb

    lax.fori_loop(0, T, body0, (z, z, z, z))

    x1 = out0_ref[...]
    b1f = b1fi_ref[...] + b1fh_ref[...]
    pf1_ref[...] = jnp.dot(x1, w1f_ref[...], preferred_element_type=jnp.float32) + b1f
    u1f = u1f_ref[...]

    def body1(t, carry):
        hf, cf = carry
        rf = pl.multiple_of(t * BP, BP)
        gf = pf1_ref[pl.ds(rf, BP), :] + jnp.dot(hf, u1f, preferred_element_type=jnp.float32)
        return _gates(gf, cf)

    hf1, _ = lax.fori_loop(0, T, body1, (z, z))

    b1b = b1bi_ref[...] + b1bh_ref[...]
    x19 = out0_ref[(T - 1) * BP:T * BP, :]
    gb1 = jnp.dot(x19, w1b_ref[...], preferred_element_type=jnp.float32) + b1b
    hb1, _ = _gates(gb1, z)

    feat = jnp.concatenate([hf1, hb1], axis=1)
    out_ref[...] = jnp.dot(feat, fcw_ref[...], preferred_element_type=jnp.float32) + fcb_ref[...]


def _lstm_call(xs, *args):
    return pl.pallas_call(
        _lstm_body,
        out_shape=jax.ShapeDtypeStruct((BP, NCLS), jnp.float32),
        scratch_shapes=[
            pltpu.VMEM((T * BP, 4 * LH), jnp.float32),
            pltpu.VMEM((T * BP, 4 * LH), jnp.float32),
            pltpu.VMEM((T * BP, 2 * LH), jnp.float32),
            pltpu.VMEM((T * BP, 4 * LH), jnp.float32),
        ],
    )(xs, *args)


def kernel(x, edge_index, W1, b1, W2, b2, W3, b3,
           Wih0f, Whh0f, bih0f, bhh0f,
           Wih0b, Whh0b, bih0b, bhh0b,
           Wih1f, Whh1f, bih1f, bhh1f,
           Wih1b, Whh1b, bih1b, bhh1b,
           fcW, fcb):
    row = edge_index[0].astype(jnp.int32)
    col = edge_index[1].astype(jnp.int32)
    padv = jnp.full((EP - E,), NP - 1, jnp.int32)
    rowp = jnp.concatenate([row, padv]).reshape(EROWS, 128)
    colf = jnp.concatenate([col, padv])
    colp = colf.reshape(EROWS, 128)
    ztbl = jnp.zeros((NP, GH), jnp.float32)

    hist = _hist_call()(colf)
    dis, yhat1 = _prep_call(hist.sum(axis=0)[:, None], x, W1)
    p1 = _agg_call()(yhat1, rowp, colp, ztbl)
    yhat2 = _mid_call(p1, dis, b1, W2)
    p2 = _agg_call()(yhat2, rowp, colp, ztbl)
    yhat3 = _mid_call(p2, dis, b2, W3)
    p3 = _agg_call()(yhat3, rowp, colp, ztbl)
    h3 = _last_call(p3, dis, b3)

    hseq = h3[:N].reshape(B, T, 5 * GH).transpose(1, 0, 2)
    hseq = jnp.pad(hseq, ((0, 0), (0, BP - B), (0, 0))).reshape(T * BP, 5 * GH)

    out = _lstm_call(
        hseq,
        Wih0f.T, Whh0f.T, bih0f, bhh0f,
        Wih0b.T, Whh0b.T, bih0b, bhh0b,
        Wih1f.T, Whh1f.T, bih1f, bhh1f,
        Wih1b.T, Whh1b.T, bih1b, bhh1b,
        fcW.T, fcb,
    )
    return out[:B]
